# baseline (speedup 1.0000x reference)
import functools

import jax
import jax.numpy as jnp
from jax import lax
from jax.experimental import pallas as pl
from jax.experimental.pallas import tpu as pltpu
from jax.experimental.pallas import tpu_sc as plsc

_ACT = 2
_LANES = 16


def _mega_kernel(h_ref, wq_ref, wk_ref, wv_ref, x_ref,
                 simt_ref, att_ref, w_s, z_s):
    b = pl.program_id(0)
    s = pl.program_id(1)
    ns = pl.num_programs(1)
    n_k = w_s.shape[0]

    @pl.when((b == 0) & (s == 0))
    def _():
        for k in range(n_k):
            q = jnp.dot(h_ref[k], wq_ref[k],
                        preferred_element_type=jnp.float32)
            w_s[pl.ds(k, 1), :] = jax.lax.dot_general(
                q, wk_ref[k], (((1,), (1,)), ((), ())),
                preferred_element_type=jnp.float32)

    w = w_s[...]
    x = x_ref[0]
    simt = jax.lax.dot_general(w, x, (((1,), (1,)), ((), ())),
                               preferred_element_type=jnp.float32)
    simt_ref[0] = simt
    zc = jnp.dot(simt, x, preferred_element_type=jnp.float32)

    @pl.when(s == 0)
    def _():
        z_s[...] = zc

    @pl.when(s > 0)
    def _():
        z_s[...] += zc

    @pl.when(s == ns - 1)
    def _():
        z = z_s[...]
        rows = [
            jnp.dot(z[k:k + 1, :], wv_ref[k],
                    preferred_element_type=jnp.float32)
            for k in range(n_k)
        ]
        att_ref[0] = jnp.concatenate(rows, axis=0)


def _make_sc_topk(B, K, H):
    mesh = plsc.VectorSubcoreMesh(core_axis_name="c", subcore_axis_name="s")
    info = plsc.get_sparse_core_info()
    nc = info.num_cores

    @functools.partial(
        pl.kernel,
        mesh=mesh,
        compiler_params=pltpu.CompilerParams(
            needs_layout_passes=False, skip_device_barrier=True),
        out_type=[
            jax.ShapeDtypeStruct((B, _LANES), jnp.float32),
            jax.ShapeDtypeStruct((B, _LANES), jnp.int32),
            jax.ShapeDtypeStruct((B * K, H), jnp.float32),
        ],
        scratch_types=[
            pltpu.VMEM((_LANES,), jnp.float32),
            pltpu.VMEM((_LANES,), jnp.float32),
            pltpu.VMEM((_LANES,), jnp.int32),
            pltpu.VMEM((H,), jnp.float32),
        ],
    )
    def sc_topk(ns_hbm, tv_hbm, ti_hbm, mask_hbm, v_v, tv_v, ti_v, m_v):
        wid = lax.axis_index("s") * nc + lax.axis_index("c")

        @pl.when(wid < B * K)
        def _():
            b = wid // K
            k = wid % K
            pltpu.sync_copy(ns_hbm.at[b], v_v)
            v = v_v[...]
            ki = lax.iota(jnp.int32, _LANES)
            def _rank_step(j, acc):
                vj = v.at[jnp.full((_LANES,), j, jnp.int32)].get(
                    mode="promise_in_bounds")
                cond = (vj < v) | ((vj == v) & (ki > j))
                return acc + cond.astype(jnp.int32)

            rank = lax.fori_loop(
                0, _LANES, _rank_step, jnp.zeros((_LANES,), jnp.int32))

            @pl.when(k == 0)
            def _():
                plsc.store_scatter(tv_v, [rank], v)
                plsc.store_scatter(ti_v, [rank], ki)
                pltpu.sync_copy(tv_v, tv_hbm.at[b])
                pltpu.sync_copy(ti_v, ti_hbm.at[b])

            sel = (rank < _ACT).astype(jnp.float32)
            splat = sel.at[jnp.full((_LANES,), k, jnp.int32)].get(
                mode="promise_in_bounds")

            def _fill_step(h, _):
                m_v[pl.ds(h * _LANES, _LANES)] = splat
                return 0

            lax.fori_loop(0, H // _LANES, _fill_step, 0)
            pltpu.sync_copy(m_v, mask_hbm.at[wid])

    return sc_topk


def kernel(input, rim_hidden, Wq, Wk, Wv):
    B, S, D = input.shape
    K, H = rim_hidden.shape
    A = Wq.shape[2]
    BS = 512
    ns = S // BS

    null_sim16 = jnp.concatenate(
        [jnp.zeros((B, K), jnp.float32),
         jnp.full((B, _LANES - K), jnp.inf, jnp.float32)], axis=1)

    tv16, ti16, mask_rows = _make_sc_topk(B, K, H)(null_sim16)
    topk_vals = tv16[:, :_ACT]
    topk_idx = ti16[:, :_ACT]
    update_mask = mask_rows.reshape(B, K, H)

    h3 = rim_hidden.reshape(K, 1, H)

    simt, att = pl.pallas_call(
        _mega_kernel,
        grid=(B, ns),
        in_specs=[
            pl.BlockSpec((K, 1, H), lambda b, s: (0, 0, 0)),
            pl.BlockSpec((K, H, A), lambda b, s: (0, 0, 0)),
            pl.BlockSpec((K, D, A), lambda b, s: (0, 0, 0)),
            pl.BlockSpec((K, D, A), lambda b, s: (0, 0, 0)),
            pl.BlockSpec((1, BS, D), lambda b, s: (b, s, 0)),
        ],
        out_specs=[
            pl.BlockSpec((1, K, BS), lambda b, s: (b, 0, s)),
            pl.BlockSpec((1, K, A), lambda b, s: (b, 0, 0)),
        ],
        out_shape=[
            jax.ShapeDtypeStruct((B, K, S), jnp.float32),
            jax.ShapeDtypeStruct((B, K, A), jnp.float32),
        ],
        scratch_shapes=[
            pltpu.VMEM((K, D), jnp.float32),
            pltpu.VMEM((K, D), jnp.float32),
        ],
    )(h3, Wq, Wk, Wv, input)

    sim = jnp.concatenate(
        [simt, jnp.zeros((B, K, 1), jnp.float32)], axis=2)

    return (att, sim, topk_vals, topk_idx, update_mask)


# speedup vs baseline: 1.0334x; 1.0334x over previous
import functools

import jax
import jax.numpy as jnp
from jax import lax
from jax.experimental import pallas as pl
from jax.experimental.pallas import tpu as pltpu
from jax.experimental.pallas import tpu_sc as plsc

_ACT = 2
_LANES = 16


def _mega_kernel(h_ref, wq_ref, wk_ref, wv_ref, x_ref,
                 simt_ref, att_ref, w_s, z_s):
    b = pl.program_id(0)
    s = pl.program_id(1)
    nb = pl.num_programs(0)
    ns = pl.num_programs(1)
    n_k = w_s.shape[0]

    @pl.when((b == 0) & (s == 0))
    def _():
        for k in range(n_k):
            q = jnp.dot(h_ref[k], wq_ref[k],
                        preferred_element_type=jnp.float32)
            w_s[pl.ds(k, 1), :] = jax.lax.dot_general(
                q, wk_ref[k], (((1,), (1,)), ((), ())),
                preferred_element_type=jnp.float32)

    w = w_s[...]
    x = x_ref[0]
    simt = jax.lax.dot_general(w, x, (((1,), (1,)), ((), ())),
                               preferred_element_type=jnp.float32)
    simt_ref[0] = simt
    zc = jnp.dot(simt, x, preferred_element_type=jnp.float32)

    @pl.when(s == 0)
    def _():
        z_s[b] = zc

    @pl.when(s > 0)
    def _():
        z_s[b] += zc

    @pl.when((b == nb - 1) & (s == ns - 1))
    def _():
        for k in range(n_k):
            att_ref[:, k, :] = jnp.dot(
                z_s[:, k, :], wv_ref[k],
                preferred_element_type=jnp.float32)


def _make_sc_topk(B, K, H):
    mesh = plsc.VectorSubcoreMesh(core_axis_name="c", subcore_axis_name="s")
    info = plsc.get_sparse_core_info()
    nc = info.num_cores

    @functools.partial(
        pl.kernel,
        mesh=mesh,
        compiler_params=pltpu.CompilerParams(
            needs_layout_passes=False, skip_device_barrier=True),
        out_type=[
            jax.ShapeDtypeStruct((B, _LANES), jnp.float32),
            jax.ShapeDtypeStruct((B, _LANES), jnp.int32),
            jax.ShapeDtypeStruct((B * K, H), jnp.float32),
        ],
        scratch_types=[
            pltpu.VMEM((_LANES,), jnp.float32),
            pltpu.VMEM((_LANES,), jnp.float32),
            pltpu.VMEM((_LANES,), jnp.int32),
            pltpu.VMEM((H,), jnp.float32),
        ],
    )
    def sc_topk(ns_hbm, tv_hbm, ti_hbm, mask_hbm, v_v, tv_v, ti_v, m_v):
        wid = lax.axis_index("s") * nc + lax.axis_index("c")

        @pl.when(wid < B * K)
        def _():
            b = wid // K
            k = wid % K
            pltpu.sync_copy(ns_hbm.at[b], v_v)
            v = v_v[...]
            ki = lax.iota(jnp.int32, _LANES)
            def _rank_step(j, acc):
                vj = v.at[jnp.full((_LANES,), j, jnp.int32)].get(
                    mode="promise_in_bounds")
                cond = (vj < v) | ((vj == v) & (ki > j))
                return acc + cond.astype(jnp.int32)

            rank = lax.fori_loop(
                0, _LANES, _rank_step, jnp.zeros((_LANES,), jnp.int32))

            @pl.when(k == 0)
            def _():
                plsc.store_scatter(tv_v, [rank], v)
                plsc.store_scatter(ti_v, [rank], ki)
                pltpu.sync_copy(tv_v, tv_hbm.at[b])
                pltpu.sync_copy(ti_v, ti_hbm.at[b])

            sel = (rank < _ACT).astype(jnp.float32)
            splat = sel.at[jnp.full((_LANES,), k, jnp.int32)].get(
                mode="promise_in_bounds")

            def _fill_step(h, _):
                m_v[pl.ds(h * _LANES, _LANES)] = splat
                return 0

            lax.fori_loop(0, H // _LANES, _fill_step, 0)
            pltpu.sync_copy(m_v, mask_hbm.at[wid])

    return sc_topk


def kernel(input, rim_hidden, Wq, Wk, Wv):
    B, S, D = input.shape
    K, H = rim_hidden.shape
    A = Wq.shape[2]
    BS = 512
    ns = S // BS

    null_sim16 = jnp.concatenate(
        [jnp.zeros((B, K), jnp.float32),
         jnp.full((B, _LANES - K), jnp.inf, jnp.float32)], axis=1)

    tv16, ti16, mask_rows = _make_sc_topk(B, K, H)(null_sim16)
    topk_vals = tv16[:, :_ACT]
    topk_idx = ti16[:, :_ACT]
    update_mask = mask_rows.reshape(B, K, H)

    h3 = rim_hidden.reshape(K, 1, H)

    simt, att = pl.pallas_call(
        _mega_kernel,
        grid=(B, ns),
        in_specs=[
            pl.BlockSpec((K, 1, H), lambda b, s: (0, 0, 0)),
            pl.BlockSpec((K, H, A), lambda b, s: (0, 0, 0)),
            pl.BlockSpec((K, D, A), lambda b, s: (0, 0, 0)),
            pl.BlockSpec((K, D, A), lambda b, s: (0, 0, 0)),
            pl.BlockSpec((1, BS, D), lambda b, s: (b, s, 0)),
        ],
        out_specs=[
            pl.BlockSpec((1, K, BS), lambda b, s: (b, 0, s)),
            pl.BlockSpec((B, K, A), lambda b, s: (0, 0, 0)),
        ],
        out_shape=[
            jax.ShapeDtypeStruct((B, K, S), jnp.float32),
            jax.ShapeDtypeStruct((B, K, A), jnp.float32),
        ],
        scratch_shapes=[
            pltpu.VMEM((K, D), jnp.float32),
            pltpu.VMEM((B, K, D), jnp.float32),
        ],
    )(h3, Wq, Wk, Wv, input)

    sim = jnp.concatenate(
        [simt, jnp.zeros((B, K, 1), jnp.float32)], axis=2)

    return (att, sim, topk_vals, topk_idx, update_mask)


# speedup vs baseline: 1.1398x; 1.1030x over previous
import functools

import jax
import jax.numpy as jnp
from jax import lax
from jax.experimental import pallas as pl
from jax.experimental.pallas import tpu as pltpu
from jax.experimental.pallas import tpu_sc as plsc

_ACT = 2
_LANES = 16


def _mega_kernel(h_ref, wq_ref, wk_ref, wv_ref, x_ref,
                 simt_ref, att_ref, w_s, z_s):
    b = pl.program_id(0)
    s = pl.program_id(1)
    nb = pl.num_programs(0)
    ns = pl.num_programs(1)
    n_k = w_s.shape[0]

    @pl.when((b == 0) & (s == 0))
    def _():
        for k in range(n_k):
            q = jnp.dot(h_ref[k], wq_ref[k],
                        preferred_element_type=jnp.float32)
            w_s[pl.ds(k, 1), :] = jax.lax.dot_general(
                q, wk_ref[k], (((1,), (1,)), ((), ())),
                preferred_element_type=jnp.float32)

    w = w_s[...]
    x = x_ref[0]
    simt = jax.lax.dot_general(w, x, (((1,), (1,)), ((), ())),
                               preferred_element_type=jnp.float32)
    simt_ref[0] = simt
    zc = jnp.dot(simt, x, preferred_element_type=jnp.float32)

    @pl.when(s == 0)
    def _():
        z_s[b] = zc

    @pl.when(s > 0)
    def _():
        z_s[b] += zc

    @pl.when((b == nb - 1) & (s == ns - 1))
    def _():
        for k in range(n_k):
            att_ref[:, k, :] = jnp.dot(
                z_s[:, k, :], wv_ref[k],
                preferred_element_type=jnp.float32)


def _make_sc_topk(B, K, H):
    mesh = plsc.VectorSubcoreMesh(core_axis_name="c", subcore_axis_name="s")
    info = plsc.get_sparse_core_info()
    nc = info.num_cores

    @functools.partial(
        pl.kernel,
        mesh=mesh,
        compiler_params=pltpu.CompilerParams(
            needs_layout_passes=False, skip_device_barrier=True),
        out_type=[
            jax.ShapeDtypeStruct((B, _LANES), jnp.float32),
            jax.ShapeDtypeStruct((B, _LANES), jnp.int32),
            jax.ShapeDtypeStruct((B * K, H), jnp.float32),
        ],
        scratch_types=[
            pltpu.VMEM((_LANES,), jnp.float32),
            pltpu.VMEM((_LANES,), jnp.float32),
            pltpu.VMEM((_LANES,), jnp.int32),
            pltpu.VMEM((H,), jnp.float32),
        ],
    )
    def sc_topk(ns_hbm, tv_hbm, ti_hbm, mask_hbm, v_v, tv_v, ti_v, m_v):
        wid = lax.axis_index("s") * nc + lax.axis_index("c")

        @pl.when(wid < B * K)
        def _():
            b = wid // K
            k = wid % K
            pltpu.sync_copy(ns_hbm.at[b], v_v)
            v = v_v[...]
            ki = lax.iota(jnp.int32, _LANES)
            def _rank_step(j, acc):
                vj = v.at[jnp.full((_LANES,), j, jnp.int32)].get(
                    mode="promise_in_bounds")
                cond = (vj < v) | ((vj == v) & (ki > j))
                return acc + cond.astype(jnp.int32)

            rank = lax.fori_loop(
                0, _LANES, _rank_step, jnp.zeros((_LANES,), jnp.int32))

            @pl.when(k == 0)
            def _():
                plsc.store_scatter(tv_v, [rank], v)
                plsc.store_scatter(ti_v, [rank], ki)
                pltpu.sync_copy(tv_v, tv_hbm.at[b])
                pltpu.sync_copy(ti_v, ti_hbm.at[b])

            sel = (rank < _ACT).astype(jnp.float32)
            splat = sel.at[jnp.full((_LANES,), k, jnp.int32)].get(
                mode="promise_in_bounds")

            def _fill_step(h, _):
                m_v[pl.ds(h * _LANES, _LANES)] = splat
                return 0

            lax.fori_loop(0, H // _LANES, _fill_step, 0)
            pltpu.sync_copy(m_v, mask_hbm.at[wid])

    return sc_topk


def kernel(input, rim_hidden, Wq, Wk, Wv):
    B, S, D = input.shape
    K, H = rim_hidden.shape
    A = Wq.shape[2]
    BS = 1024
    ns = S // BS

    null_sim16 = jnp.concatenate(
        [jnp.zeros((B, K), jnp.float32),
         jnp.full((B, _LANES - K), jnp.inf, jnp.float32)], axis=1)

    tv16, ti16, mask_rows = _make_sc_topk(B, K, H)(null_sim16)
    topk_vals = tv16[:, :_ACT]
    topk_idx = ti16[:, :_ACT]
    update_mask = mask_rows.reshape(B, K, H)

    h3 = rim_hidden.reshape(K, 1, H)

    simt, att = pl.pallas_call(
        _mega_kernel,
        grid=(B, ns),
        in_specs=[
            pl.BlockSpec((K, 1, H), lambda b, s: (0, 0, 0)),
            pl.BlockSpec((K, H, A), lambda b, s: (0, 0, 0)),
            pl.BlockSpec((K, D, A), lambda b, s: (0, 0, 0)),
            pl.BlockSpec((K, D, A), lambda b, s: (0, 0, 0)),
            pl.BlockSpec((1, BS, D), lambda b, s: (b, s, 0)),
        ],
        out_specs=[
            pl.BlockSpec((1, K, BS), lambda b, s: (b, 0, s)),
            pl.BlockSpec((B, K, A), lambda b, s: (0, 0, 0)),
        ],
        out_shape=[
            jax.ShapeDtypeStruct((B, K, S), jnp.float32),
            jax.ShapeDtypeStruct((B, K, A), jnp.float32),
        ],
        scratch_shapes=[
            pltpu.VMEM((K, D), jnp.float32),
            pltpu.VMEM((B, K, D), jnp.float32),
        ],
    )(h3, Wq, Wk, Wv, input)

    sim = jnp.concatenate(
        [simt, jnp.zeros((B, K, 1), jnp.float32)], axis=2)

    return (att, sim, topk_vals, topk_idx, update_mask)


# speedup vs baseline: 1.1749x; 1.0307x over previous
import functools

import jax
import jax.numpy as jnp
from jax import lax
from jax.experimental import pallas as pl
from jax.experimental.pallas import tpu as pltpu
from jax.experimental.pallas import tpu_sc as plsc

_ACT = 2
_LANES = 16


def _mega_kernel(h_ref, wq_ref, wk_ref, wv_ref, x_ref,
                 simt_ref, att_ref, w_s, z_s):
    b = pl.program_id(0)
    s = pl.program_id(1)
    nb = pl.num_programs(0)
    ns = pl.num_programs(1)
    n_k = w_s.shape[0]

    @pl.when((b == 0) & (s == 0))
    def _():
        for k in range(n_k):
            q = jnp.dot(h_ref[k], wq_ref[k],
                        preferred_element_type=jnp.float32)
            w_s[pl.ds(k, 1), :] = jax.lax.dot_general(
                q, wk_ref[k], (((1,), (1,)), ((), ())),
                preferred_element_type=jnp.float32)

    w = w_s[...]
    x = x_ref[0]
    simt = jax.lax.dot_general(w, x, (((1,), (1,)), ((), ())),
                               preferred_element_type=jnp.float32)
    simt_ref[0] = simt
    zc = jnp.dot(simt, x, preferred_element_type=jnp.float32)

    @pl.when(s == 0)
    def _():
        z_s[b] = zc

    @pl.when(s > 0)
    def _():
        z_s[b] += zc

    @pl.when((b == nb - 1) & (s == ns - 1))
    def _():
        for k in range(n_k):
            att_ref[:, k, :] = jnp.dot(
                z_s[:, k, :], wv_ref[k],
                preferred_element_type=jnp.float32)


def _make_sc_topk(B, K, H):
    mesh = plsc.VectorSubcoreMesh(core_axis_name="c", subcore_axis_name="s")
    info = plsc.get_sparse_core_info()
    nc = info.num_cores

    @functools.partial(
        pl.kernel,
        mesh=mesh,
        compiler_params=pltpu.CompilerParams(
            needs_layout_passes=False, skip_device_barrier=True),
        out_type=[
            jax.ShapeDtypeStruct((B, _LANES), jnp.float32),
            jax.ShapeDtypeStruct((B, _LANES), jnp.int32),
            jax.ShapeDtypeStruct((B * K, H), jnp.float32),
        ],
        scratch_types=[
            pltpu.VMEM((_LANES,), jnp.float32),
            pltpu.VMEM((_LANES,), jnp.float32),
            pltpu.VMEM((_LANES,), jnp.int32),
            pltpu.VMEM((H,), jnp.float32),
        ],
    )
    def sc_topk(ns_hbm, tv_hbm, ti_hbm, mask_hbm, v_v, tv_v, ti_v, m_v):
        wid = lax.axis_index("s") * nc + lax.axis_index("c")

        @pl.when(wid < B * K)
        def _():
            b = wid // K
            k = wid % K
            pltpu.sync_copy(ns_hbm.at[b], v_v)
            v = v_v[...]
            ki = lax.iota(jnp.int32, _LANES)
            def _rank_step(j, acc):
                vj = v.at[jnp.full((_LANES,), j, jnp.int32)].get(
                    mode="promise_in_bounds")
                cond = (vj < v) | ((vj == v) & (ki > j))
                return acc + cond.astype(jnp.int32)

            rank = lax.fori_loop(
                0, _LANES, _rank_step, jnp.zeros((_LANES,), jnp.int32))

            @pl.when(k == 0)
            def _():
                plsc.store_scatter(tv_v, [rank], v)
                plsc.store_scatter(ti_v, [rank], ki)
                pltpu.sync_copy(tv_v, tv_hbm.at[b])
                pltpu.sync_copy(ti_v, ti_hbm.at[b])

            sel = (rank < _ACT).astype(jnp.float32)
            splat = sel.at[jnp.full((_LANES,), k, jnp.int32)].get(
                mode="promise_in_bounds")

            def _fill_step(h, _):
                m_v[pl.ds(h * _LANES, _LANES)] = splat
                return 0

            lax.fori_loop(0, H // _LANES, _fill_step, 0)
            pltpu.sync_copy(m_v, mask_hbm.at[wid])

    return sc_topk


def kernel(input, rim_hidden, Wq, Wk, Wv):
    B, S, D = input.shape
    K, H = rim_hidden.shape
    A = Wq.shape[2]
    BS = 2048
    ns = S // BS

    null_sim16 = jnp.concatenate(
        [jnp.zeros((B, K), jnp.float32),
         jnp.full((B, _LANES - K), jnp.inf, jnp.float32)], axis=1)

    tv16, ti16, mask_rows = _make_sc_topk(B, K, H)(null_sim16)
    topk_vals = tv16[:, :_ACT]
    topk_idx = ti16[:, :_ACT]
    update_mask = mask_rows.reshape(B, K, H)

    h3 = rim_hidden.reshape(K, 1, H)

    simt, att = pl.pallas_call(
        _mega_kernel,
        grid=(B, ns),
        in_specs=[
            pl.BlockSpec((K, 1, H), lambda b, s: (0, 0, 0)),
            pl.BlockSpec((K, H, A), lambda b, s: (0, 0, 0)),
            pl.BlockSpec((K, D, A), lambda b, s: (0, 0, 0)),
            pl.BlockSpec((K, D, A), lambda b, s: (0, 0, 0)),
            pl.BlockSpec((1, BS, D), lambda b, s: (b, s, 0)),
        ],
        out_specs=[
            pl.BlockSpec((1, K, BS), lambda b, s: (b, 0, s)),
            pl.BlockSpec((B, K, A), lambda b, s: (0, 0, 0)),
        ],
        out_shape=[
            jax.ShapeDtypeStruct((B, K, S), jnp.float32),
            jax.ShapeDtypeStruct((B, K, A), jnp.float32),
        ],
        scratch_shapes=[
            pltpu.VMEM((K, D), jnp.float32),
            pltpu.VMEM((B, K, D), jnp.float32),
        ],
    )(h3, Wq, Wk, Wv, input)

    sim = jnp.concatenate(
        [simt, jnp.zeros((B, K, 1), jnp.float32)], axis=2)

    return (att, sim, topk_vals, topk_idx, update_mask)


# speedup vs baseline: 1.1754x; 1.0004x over previous
import functools

import jax
import jax.numpy as jnp
from jax import lax
from jax.experimental import pallas as pl
from jax.experimental.pallas import tpu as pltpu
from jax.experimental.pallas import tpu_sc as plsc

_ACT = 2
_LANES = 16


def _mega_kernel(h_ref, wq_ref, wk_ref, wv_ref, x_ref,
                 simt_ref, att_ref, w_s, z_s):
    b = pl.program_id(0)
    s = pl.program_id(1)
    nb = pl.num_programs(0)
    ns = pl.num_programs(1)
    n_k = w_s.shape[0]

    @pl.when((b == 0) & (s == 0))
    def _():
        for k in range(n_k):
            q = jnp.dot(h_ref[k], wq_ref[k],
                        preferred_element_type=jnp.float32)
            w_s[pl.ds(k, 1), :] = jax.lax.dot_general(
                q, wk_ref[k], (((1,), (1,)), ((), ())),
                preferred_element_type=jnp.float32)

    w = w_s[...]
    x = x_ref[0]
    simt = jax.lax.dot_general(w, x, (((1,), (1,)), ((), ())),
                               preferred_element_type=jnp.float32)
    simt_ref[0] = simt
    zc = jnp.dot(simt, x, preferred_element_type=jnp.float32)

    @pl.when(s == 0)
    def _():
        z_s[b] = zc

    @pl.when(s > 0)
    def _():
        z_s[b] += zc

    @pl.when((b == nb - 1) & (s == ns - 1))
    def _():
        for k in range(n_k):
            att_ref[:, k, :] = jnp.dot(
                z_s[:, k, :], wv_ref[k],
                preferred_element_type=jnp.float32)


def _make_sc_topk(B, K, H):
    mesh = plsc.VectorSubcoreMesh(core_axis_name="c", subcore_axis_name="s")
    info = plsc.get_sparse_core_info()
    nc = info.num_cores

    @functools.partial(
        pl.kernel,
        mesh=mesh,
        compiler_params=pltpu.CompilerParams(needs_layout_passes=False),
        out_type=[
            jax.ShapeDtypeStruct((B, _LANES), jnp.float32),
            jax.ShapeDtypeStruct((B, _LANES), jnp.int32),
            jax.ShapeDtypeStruct((B * K, H), jnp.float32),
        ],
        scratch_types=[
            pltpu.VMEM((_LANES,), jnp.float32),
            pltpu.VMEM((_LANES,), jnp.float32),
            pltpu.VMEM((_LANES,), jnp.int32),
            pltpu.VMEM((H,), jnp.float32),
        ],
    )
    def sc_topk(ns_hbm, tv_hbm, ti_hbm, mask_hbm, v_v, tv_v, ti_v, m_v):
        wid = lax.axis_index("s") * nc + lax.axis_index("c")

        @pl.when(wid < B * K)
        def _():
            b = wid // K
            k = wid % K
            pltpu.sync_copy(ns_hbm.at[b], v_v)
            v = v_v[...]
            ki = lax.iota(jnp.int32, _LANES)
            def _rank_step(j, acc):
                vj = v.at[jnp.full((_LANES,), j, jnp.int32)].get(
                    mode="promise_in_bounds")
                cond = (vj < v) | ((vj == v) & (ki > j))
                return acc + cond.astype(jnp.int32)

            rank = lax.fori_loop(
                0, _LANES, _rank_step, jnp.zeros((_LANES,), jnp.int32))

            @pl.when(k == 0)
            def _():
                plsc.store_scatter(tv_v, [rank], v)
                plsc.store_scatter(ti_v, [rank], ki)
                pltpu.sync_copy(tv_v, tv_hbm.at[b])
                pltpu.sync_copy(ti_v, ti_hbm.at[b])

            sel = (rank < _ACT).astype(jnp.float32)
            splat = sel.at[jnp.full((_LANES,), k, jnp.int32)].get(
                mode="promise_in_bounds")

            def _fill_step(h, _):
                m_v[pl.ds(h * _LANES, _LANES)] = splat
                return 0

            lax.fori_loop(0, H // _LANES, _fill_step, 0)
            pltpu.sync_copy(m_v, mask_hbm.at[wid])

    return sc_topk


def kernel(input, rim_hidden, Wq, Wk, Wv):
    B, S, D = input.shape
    K, H = rim_hidden.shape
    A = Wq.shape[2]
    BS = 2048
    ns = S // BS

    null_sim16 = jnp.concatenate(
        [jnp.zeros((B, K), jnp.float32),
         jnp.full((B, _LANES - K), jnp.inf, jnp.float32)], axis=1)

    tv16, ti16, mask_rows = _make_sc_topk(B, K, H)(null_sim16)
    topk_vals = tv16[:, :_ACT]
    topk_idx = ti16[:, :_ACT]
    update_mask = mask_rows.reshape(B, K, H)

    h3 = rim_hidden.reshape(K, 1, H)

    simt, att = pl.pallas_call(
        _mega_kernel,
        grid=(B, ns),
        in_specs=[
            pl.BlockSpec((K, 1, H), lambda b, s: (0, 0, 0)),
            pl.BlockSpec((K, H, A), lambda b, s: (0, 0, 0)),
            pl.BlockSpec((K, D, A), lambda b, s: (0, 0, 0)),
            pl.BlockSpec((K, D, A), lambda b, s: (0, 0, 0)),
            pl.BlockSpec((1, BS, D), lambda b, s: (b, s, 0)),
        ],
        out_specs=[
            pl.BlockSpec((1, K, BS), lambda b, s: (b, 0, s)),
            pl.BlockSpec((B, K, A), lambda b, s: (0, 0, 0)),
        ],
        out_shape=[
            jax.ShapeDtypeStruct((B, K, S), jnp.float32),
            jax.ShapeDtypeStruct((B, K, A), jnp.float32),
        ],
        scratch_shapes=[
            pltpu.VMEM((K, D), jnp.float32),
            pltpu.VMEM((B, K, D), jnp.float32),
        ],
    )(h3, Wq, Wk, Wv, input)

    sim = jnp.concatenate(
        [simt, jnp.zeros((B, K, 1), jnp.float32)], axis=2)

    return (att, sim, topk_vals, topk_idx, update_mask)


# speedup vs baseline: 1.2141x; 1.0329x over previous
import functools

import jax
import jax.numpy as jnp
from jax import lax
from jax.experimental import pallas as pl
from jax.experimental.pallas import tpu as pltpu
from jax.experimental.pallas import tpu_sc as plsc

_ACT = 2
_LANES = 16


def _mega_kernel(h_ref, wq_ref, wk_ref, wv_ref, x_ref,
                 simt_ref, att_ref, w_s, z_s):
    b = pl.program_id(0)
    s = pl.program_id(1)
    nb = pl.num_programs(0)
    ns = pl.num_programs(1)
    n_k = w_s.shape[0]

    @pl.when((b == 0) & (s == 0))
    def _():
        for k in range(n_k):
            q = jnp.dot(h_ref[k], wq_ref[k],
                        preferred_element_type=jnp.float32)
            w_s[pl.ds(k, 1), :] = jax.lax.dot_general(
                q, wk_ref[k], (((1,), (1,)), ((), ())),
                preferred_element_type=jnp.float32)

    w = w_s[...]
    x = x_ref[0]
    simt = jax.lax.dot_general(w, x, (((1,), (1,)), ((), ())),
                               preferred_element_type=jnp.float32)
    simt_ref[0] = simt
    zc = jnp.dot(simt, x, preferred_element_type=jnp.float32)

    @pl.when(s == 0)
    def _():
        z_s[b] = zc

    @pl.when(s > 0)
    def _():
        z_s[b] += zc

    @pl.when((b == nb - 1) & (s == ns - 1))
    def _():
        for k in range(n_k):
            att_ref[:, k, :] = jnp.dot(
                z_s[:, k, :], wv_ref[k],
                preferred_element_type=jnp.float32)


def _make_sc_topk(B, K, H):
    mesh = plsc.VectorSubcoreMesh(
        core_axis_name="c", subcore_axis_name="s", num_cores=1)

    @functools.partial(
        pl.kernel,
        mesh=mesh,
        compiler_params=pltpu.CompilerParams(needs_layout_passes=False),
        out_type=[
            jax.ShapeDtypeStruct((B, _LANES), jnp.float32),
            jax.ShapeDtypeStruct((B, _LANES), jnp.int32),
            jax.ShapeDtypeStruct((B * K, H), jnp.float32),
        ],
        scratch_types=[
            pltpu.VMEM((_LANES,), jnp.float32),
            pltpu.VMEM((_LANES,), jnp.float32),
            pltpu.VMEM((_LANES,), jnp.int32),
            pltpu.VMEM((H,), jnp.float32),
        ],
    )
    def sc_topk(ns_hbm, tv_hbm, ti_hbm, mask_hbm, v_v, tv_v, ti_v, m_v):
        sid = lax.axis_index("s")
        n_sub = 16

        def _row(wid):
            b = wid // K
            k = wid % K
            pltpu.sync_copy(ns_hbm.at[b], v_v)
            v = v_v[...]
            ki = lax.iota(jnp.int32, _LANES)
            def _rank_step(j, acc):
                vj = v.at[jnp.full((_LANES,), j, jnp.int32)].get(
                    mode="promise_in_bounds")
                cond = (vj < v) | ((vj == v) & (ki > j))
                return acc + cond.astype(jnp.int32)

            rank = lax.fori_loop(
                0, _LANES, _rank_step, jnp.zeros((_LANES,), jnp.int32))

            @pl.when(k == 0)
            def _():
                plsc.store_scatter(tv_v, [rank], v)
                plsc.store_scatter(ti_v, [rank], ki)
                pltpu.sync_copy(tv_v, tv_hbm.at[b])
                pltpu.sync_copy(ti_v, ti_hbm.at[b])

            sel = (rank < _ACT).astype(jnp.float32)
            splat = sel.at[jnp.full((_LANES,), k, jnp.int32)].get(
                mode="promise_in_bounds")

            def _fill_step(h, _):
                m_v[pl.ds(h * _LANES, _LANES)] = splat
                return 0

            lax.fori_loop(0, H // _LANES, _fill_step, 0)
            pltpu.sync_copy(m_v, mask_hbm.at[wid])

        for r in range((B * K + n_sub - 1) // n_sub):
            _row(sid + r * n_sub)

    return sc_topk


def kernel(input, rim_hidden, Wq, Wk, Wv):
    B, S, D = input.shape
    K, H = rim_hidden.shape
    A = Wq.shape[2]
    BS = 2048
    ns = S // BS

    null_sim16 = jnp.concatenate(
        [jnp.zeros((B, K), jnp.float32),
         jnp.full((B, _LANES - K), jnp.inf, jnp.float32)], axis=1)

    tv16, ti16, mask_rows = _make_sc_topk(B, K, H)(null_sim16)
    topk_vals = tv16[:, :_ACT]
    topk_idx = ti16[:, :_ACT]
    update_mask = mask_rows.reshape(B, K, H)

    h3 = rim_hidden.reshape(K, 1, H)

    simt, att = pl.pallas_call(
        _mega_kernel,
        grid=(B, ns),
        in_specs=[
            pl.BlockSpec((K, 1, H), lambda b, s: (0, 0, 0)),
            pl.BlockSpec((K, H, A), lambda b, s: (0, 0, 0)),
            pl.BlockSpec((K, D, A), lambda b, s: (0, 0, 0)),
            pl.BlockSpec((K, D, A), lambda b, s: (0, 0, 0)),
            pl.BlockSpec((1, BS, D), lambda b, s: (b, s, 0)),
        ],
        out_specs=[
            pl.BlockSpec((1, K, BS), lambda b, s: (b, 0, s)),
            pl.BlockSpec((B, K, A), lambda b, s: (0, 0, 0)),
        ],
        out_shape=[
            jax.ShapeDtypeStruct((B, K, S), jnp.float32),
            jax.ShapeDtypeStruct((B, K, A), jnp.float32),
        ],
        scratch_shapes=[
            pltpu.VMEM((K, D), jnp.float32),
            pltpu.VMEM((B, K, D), jnp.float32),
        ],
    )(h3, Wq, Wk, Wv, input)

    sim = jnp.concatenate(
        [simt, jnp.zeros((B, K, 1), jnp.float32)], axis=2)

    return (att, sim, topk_vals, topk_idx, update_mask)
